# baseline (speedup 1.0000x reference)
import functools

import jax
import jax.numpy as jnp
from jax import lax
from jax.experimental import pallas as pl
from jax.experimental.pallas import tpu as pltpu
from jax.experimental.pallas import tpu_sc as plsc

ROWS = 32
CW = 1024

SC_WORKERS = 32
RPW = 3
TC_ROWS = 256 - SC_WORKERS * RPW
CHUNK = 20000
GR = 2000


def _scan_lanes(x, width, lanes):
    sh = 1
    while sh < width:
        rolled = pltpu.roll(x, sh, axis=1)
        x = x + jnp.where(lanes >= sh, rolled, 0.0)
        sh *= 2
    return x


def _tc_body(nchunks, vsize, p_ref, rng_ref, out_ref, sums_ref):
    rngv = rng_ref[0]

    for c in range(nchunks):
        w = min(CW, vsize - c * CW)
        chunk = p_ref[:, c * CW:c * CW + w]
        sums_ref[:, c:c + 1] = jnp.sum(chunk, axis=1, keepdims=True)

    sums = sums_ref[...]
    clanes = jax.lax.broadcasted_iota(jnp.int32, (ROWS, 128), 1)
    incl = _scan_lanes(jnp.where(clanes < nchunks, sums, 0.0), 128, clanes)
    below = jnp.logical_and(incl < rngv, clanes < nchunks)
    widths = jnp.minimum(
        jnp.maximum(vsize - clanes * CW, 0), CW)
    base = jnp.sum(jnp.where(below, widths, 0), axis=1, keepdims=True)
    pstar = jnp.sum(jnp.where(below, sums, 0.0), axis=1, keepdims=True)
    cstar = jnp.sum(jnp.where(below, 1, 0), axis=1, keepdims=True)

    cand = jnp.zeros((ROWS, CW), jnp.float32)
    for c in range(nchunks):
        w = min(CW, vsize - c * CW)
        chunk = p_ref[:, c * CW:c * CW + w]
        if w < CW:
            chunk = jnp.concatenate(
                [chunk, jnp.zeros((ROWS, CW - w), jnp.float32)], axis=1)
        cand = jnp.where(cstar == c, chunk, cand)

    lanes = jax.lax.broadcasted_iota(jnp.int32, (ROWS, CW), 1)
    lc = _scan_lanes(cand, CW, lanes)
    cwidth = jnp.where(cstar >= nchunks, 0,
                       jnp.minimum(vsize - cstar * CW, CW))
    valid = jnp.logical_and(lanes < cwidth, pstar + lc < rngv)
    cnt = jnp.sum(jnp.where(valid, 1, 0), axis=1, keepdims=True)
    out_ref[0] = base + cnt


def _tc_sample(p2, rng, vsize):
    nchunks = -(-vsize // CW)
    rng3 = rng.reshape(-1)[:TC_ROWS].reshape(TC_ROWS // ROWS, ROWS, 1)
    out = pl.pallas_call(
        functools.partial(_tc_body, nchunks, vsize),
        grid=(TC_ROWS // ROWS,),
        in_specs=[
            pl.BlockSpec((ROWS, vsize), lambda i: (i, 0)),
            pl.BlockSpec((1, ROWS, 1), lambda i: (i, 0, 0)),
        ],
        out_specs=pl.BlockSpec((1, ROWS, 1), lambda i: (i, 0, 0)),
        out_shape=jax.ShapeDtypeStruct((TC_ROWS // ROWS, ROWS, 1), jnp.int32),
        scratch_shapes=[pltpu.VMEM((ROWS, 128), jnp.float32)],
        compiler_params=pltpu.CompilerParams(
            dimension_semantics=("arbitrary",)),
    )(p2, rng3)
    return out.reshape(TC_ROWS)


def _sc_body(vsize, p_flat, rng_hbm, out_hbm,
             buf0, buf1, rngbuf, red, samp, sem0, sem1):
    nchunk = vsize // CHUNK
    ng = CHUNK // GR
    nv = GR // 16
    wid = lax.axis_index("s") * 2 + lax.axis_index("c")
    base_row = TC_ROWS + wid * RPW
    astart = (base_row // 8) * 8
    pltpu.sync_copy(rng_hbm.at[pl.ds(astart, 16)], rngbuf)
    rngwin = rngbuf[...]

    zer16 = jnp.zeros((16,), jnp.float32)
    red[pl.ds(0, 16)] = zer16
    red[pl.ds(32, 16)] = zer16

    def vtotal(v):
        s = v
        for sh in (8, 4, 2, 1):
            red[pl.ds(16, 16)] = s
            s = s + red[pl.ds(16 + sh, 16)]
        return s[0]

    def vprefix(v):
        pv = v
        for sh in (1, 2, 4, 8):
            red[pl.ds(16, 16)] = pv
            pv = pv + red[pl.ds(16 - sh, 16)]
        return pv

    bufs = (buf0, buf1)
    sems = (sem0, sem1)
    pairs = [(k, c) for k in range(RPW) for c in range(nchunk)]

    def src(k, c):
        return p_flat.at[
            pl.ds((base_row + k) * vsize + c * CHUNK, CHUNK)]

    handles = {0: pltpu.async_copy(src(*pairs[0]), bufs[0], sems[0])}

    sampv = jnp.zeros((16,), jnp.int32)
    lane16 = lax.iota(jnp.int32, 16)
    row_state = (jnp.float32(0.0), jnp.int32(0), jnp.float32(0.0))
    rngr = jnp.float32(0.0)

    for i, (k, c) in enumerate(pairs):
        buf = bufs[i % 2]
        handles[i].wait()
        if i + 1 < len(pairs):
            handles[i + 1] = pltpu.async_copy(
                src(*pairs[i + 1]), bufs[(i + 1) % 2], sems[(i + 1) % 2])

        if c == 0:
            d = base_row - astart + k
            rngr = vtotal(jnp.where(lane16 == d, rngwin, 0.0))
            row_state = (jnp.float32(0.0), jnp.int32(0), jnp.float32(0.0))

        def gbody(g, st, buf=buf, rngr=rngr):
            carry, basec, cntf = st
            off = g * GR

            def vbody(j, acc):
                return acc + buf[pl.ds(off + j * 16, 16)]

            acc = lax.fori_loop(0, nv, vbody, zer16)
            gsum = vtotal(acc)
            newc = carry + gsum
            below = newc < rngr
            is_b = jnp.logical_and(carry < rngr, jnp.logical_not(below))
            basec = basec + jnp.where(below, GR, 0)

            def fbody(j, st2):
                cntv, cb = st2
                v = buf[pl.ds(off + j * 16, 16)]
                pf = vprefix(v)
                cond = cb + pf < rngr
                cntv = cntv + jnp.where(cond, 1.0, 0.0)
                cb = cb + pf[15]
                return cntv, cb

            def fine(_):
                cntv, _ = lax.fori_loop(
                    0, nv, fbody, (zer16, carry + jnp.float32(0.0)))
                return vtotal(cntv)

            cntf = cntf + lax.cond(
                is_b, fine, lambda _: jnp.float32(0.0), 0)
            return newc, basec, cntf

        row_state = lax.fori_loop(0, ng, gbody, row_state)

        if c == nchunk - 1:
            carry, basec, cntf = row_state
            sample = basec + cntf.astype(jnp.int32)
            sampv = jnp.where(lane16 == k, sample, sampv)
            row_state = (jnp.float32(0.0), jnp.int32(0), jnp.float32(0.0))

    samp[...] = sampv
    pltpu.sync_copy(samp, out_hbm.at[wid])


def _sc_sample(p2, rng, vsize):
    mesh = plsc.VectorSubcoreMesh(core_axis_name="c", subcore_axis_name="s")
    kern = functools.partial(
        pl.kernel, mesh=mesh,
        out_type=jax.ShapeDtypeStruct((SC_WORKERS, 16), jnp.int32),
        scratch_types=[
            pltpu.VMEM((CHUNK,), jnp.float32),
            pltpu.VMEM((CHUNK,), jnp.float32),
            pltpu.VMEM((16,), jnp.float32),
            pltpu.VMEM((48,), jnp.float32),
            pltpu.VMEM((16,), jnp.int32),
            pltpu.SemaphoreType.DMA,
            pltpu.SemaphoreType.DMA,
        ],
    )(functools.partial(_sc_body, vsize))
    out = kern(p2.reshape(-1), rng.reshape(-1))
    return out[:, :RPW].reshape(SC_WORKERS * RPW)


@jax.jit
def kernel(p, rng):
    B, T, V = p.shape
    p2 = p.reshape(B * T, V)
    tc = _tc_sample(p2, rng, V)
    sc = _sc_sample(p2, rng, V)
    return jnp.concatenate([tc, sc]).reshape(B, T)


# speedup vs baseline: 1.3059x; 1.3059x over previous
import functools

import jax
import jax.numpy as jnp
from jax import lax
from jax.experimental import pallas as pl
from jax.experimental.pallas import tpu as pltpu
from jax.experimental.pallas import tpu_sc as plsc

ROWS = 32
CW = 1024

SC_WORKERS = 32
RPW = 2
TC_ROWS = 256 - SC_WORKERS * RPW
CHUNK = 20000
GR = 2000


def _scan_lanes(x, width, lanes):
    sh = 1
    while sh < width:
        rolled = pltpu.roll(x, sh, axis=1)
        x = x + jnp.where(lanes >= sh, rolled, 0.0)
        sh *= 2
    return x


def _tc_body(nchunks, vsize, p_ref, rng_ref, out_ref, sums_ref):
    rngv = rng_ref[0]

    for c in range(nchunks):
        w = min(CW, vsize - c * CW)
        chunk = p_ref[:, c * CW:c * CW + w]
        sums_ref[:, c:c + 1] = jnp.sum(chunk, axis=1, keepdims=True)

    sums = sums_ref[...]
    clanes = jax.lax.broadcasted_iota(jnp.int32, (ROWS, 128), 1)
    incl = _scan_lanes(jnp.where(clanes < nchunks, sums, 0.0), 128, clanes)
    below = jnp.logical_and(incl < rngv, clanes < nchunks)
    widths = jnp.minimum(
        jnp.maximum(vsize - clanes * CW, 0), CW)
    base = jnp.sum(jnp.where(below, widths, 0), axis=1, keepdims=True)
    pstar = jnp.sum(jnp.where(below, sums, 0.0), axis=1, keepdims=True)
    cstar = jnp.sum(jnp.where(below, 1, 0), axis=1, keepdims=True)

    cand = jnp.zeros((ROWS, CW), jnp.float32)
    for c in range(nchunks):
        w = min(CW, vsize - c * CW)
        chunk = p_ref[:, c * CW:c * CW + w]
        if w < CW:
            chunk = jnp.concatenate(
                [chunk, jnp.zeros((ROWS, CW - w), jnp.float32)], axis=1)
        cand = jnp.where(cstar == c, chunk, cand)

    lanes = jax.lax.broadcasted_iota(jnp.int32, (ROWS, CW), 1)
    lc = _scan_lanes(cand, CW, lanes)
    cwidth = jnp.where(cstar >= nchunks, 0,
                       jnp.minimum(vsize - cstar * CW, CW))
    valid = jnp.logical_and(lanes < cwidth, pstar + lc < rngv)
    cnt = jnp.sum(jnp.where(valid, 1, 0), axis=1, keepdims=True)
    out_ref[0] = base + cnt


def _tc_sample(p2, rng, vsize):
    nchunks = -(-vsize // CW)
    rng3 = rng.reshape(-1)[:TC_ROWS].reshape(TC_ROWS // ROWS, ROWS, 1)
    out = pl.pallas_call(
        functools.partial(_tc_body, nchunks, vsize),
        grid=(TC_ROWS // ROWS,),
        in_specs=[
            pl.BlockSpec((ROWS, vsize), lambda i: (i, 0)),
            pl.BlockSpec((1, ROWS, 1), lambda i: (i, 0, 0)),
        ],
        out_specs=pl.BlockSpec((1, ROWS, 1), lambda i: (i, 0, 0)),
        out_shape=jax.ShapeDtypeStruct((TC_ROWS // ROWS, ROWS, 1), jnp.int32),
        scratch_shapes=[pltpu.VMEM((ROWS, 128), jnp.float32)],
        compiler_params=pltpu.CompilerParams(
            dimension_semantics=("arbitrary",)),
    )(p2, rng3)
    return out.reshape(TC_ROWS)


def _sc_body(vsize, p_flat, rng_hbm, out_hbm,
             buf0, buf1, rngbuf, red, samp, sem0, sem1):
    nchunk = vsize // CHUNK
    ng = CHUNK // GR
    nv = GR // 16
    wid = lax.axis_index("s") * 2 + lax.axis_index("c")
    base_row = TC_ROWS + wid * RPW
    astart = (base_row // 8) * 8
    pltpu.sync_copy(rng_hbm.at[pl.ds(astart, 16)], rngbuf)
    rngwin = rngbuf[...]

    zer16 = jnp.zeros((16,), jnp.float32)
    red[pl.ds(0, 16)] = zer16
    red[pl.ds(32, 16)] = zer16

    def vtotal(v):
        s = v
        for sh in (8, 4, 2, 1):
            red[pl.ds(16, 16)] = s
            s = s + red[pl.ds(16 + sh, 16)]
        return s[0]

    def vprefix(v):
        pv = v
        for sh in (1, 2, 4, 8):
            red[pl.ds(16, 16)] = pv
            pv = pv + red[pl.ds(16 - sh, 16)]
        return pv

    bufs = (buf0, buf1)
    sems = (sem0, sem1)
    pairs = [(k, c) for k in range(RPW) for c in range(nchunk)]

    def src(k, c):
        return p_flat.at[
            pl.ds((base_row + k) * vsize + c * CHUNK, CHUNK)]

    handles = {0: pltpu.async_copy(src(*pairs[0]), bufs[0], sems[0])}

    sampv = jnp.zeros((16,), jnp.int32)
    lane16 = lax.iota(jnp.int32, 16)
    row_state = (jnp.float32(0.0), jnp.int32(0), jnp.float32(0.0))
    rngr = jnp.float32(0.0)

    for i, (k, c) in enumerate(pairs):
        buf = bufs[i % 2]
        handles[i].wait()
        if i + 1 < len(pairs):
            handles[i + 1] = pltpu.async_copy(
                src(*pairs[i + 1]), bufs[(i + 1) % 2], sems[(i + 1) % 2])

        if c == 0:
            d = base_row - astart + k
            rngr = vtotal(jnp.where(lane16 == d, rngwin, 0.0))
            row_state = (jnp.float32(0.0), jnp.int32(0), jnp.float32(0.0))

        def gbody(g, st, buf=buf, rngr=rngr):
            carry, basec, cntf = st
            off = g * GR

            def vbody(t, acc, buf=buf):
                off2 = off + t * 400
                parts = []
                for q in range(5):
                    s = buf[pl.ds(off2 + q * 80, 16)]
                    for m in range(1, 5):
                        s = s + buf[pl.ds(off2 + q * 80 + m * 16, 16)]
                    parts.append(s)
                return acc + ((parts[0] + parts[1])
                              + (parts[2] + parts[3]) + parts[4])

            acc = lax.fori_loop(0, nv // 25, vbody, zer16)
            gsum = vtotal(acc)
            newc = carry + gsum
            below = newc < rngr
            is_b = jnp.logical_and(carry < rngr, jnp.logical_not(below))
            basec = basec + jnp.where(below, GR, 0)

            def fbody(t, st2, buf=buf):
                cntv, cb = st2
                off2 = off + t * 80
                for m in range(5):
                    v = buf[pl.ds(off2 + m * 16, 16)]
                    pf = vprefix(v)
                    cntv = cntv + jnp.where(cb + pf < rngr, 1.0, 0.0)
                    cb = cb + pf[15]
                return cntv, cb

            def fine(_):
                cntv, _ = lax.fori_loop(
                    0, nv // 5, fbody, (zer16, carry + jnp.float32(0.0)))
                return vtotal(cntv)

            cntf = cntf + lax.cond(
                is_b, fine, lambda _: jnp.float32(0.0), 0)
            return newc, basec, cntf

        row_state = lax.fori_loop(0, ng, gbody, row_state)

        if c == nchunk - 1:
            carry, basec, cntf = row_state
            sample = basec + cntf.astype(jnp.int32)
            sampv = jnp.where(lane16 == k, sample, sampv)
            row_state = (jnp.float32(0.0), jnp.int32(0), jnp.float32(0.0))

    samp[...] = sampv
    pltpu.sync_copy(samp, out_hbm.at[wid])


def _sc_sample(p2, rng, vsize):
    mesh = plsc.VectorSubcoreMesh(core_axis_name="c", subcore_axis_name="s")
    kern = functools.partial(
        pl.kernel, mesh=mesh,
        out_type=jax.ShapeDtypeStruct((SC_WORKERS, 16), jnp.int32),
        scratch_types=[
            pltpu.VMEM((CHUNK,), jnp.float32),
            pltpu.VMEM((CHUNK,), jnp.float32),
            pltpu.VMEM((16,), jnp.float32),
            pltpu.VMEM((48,), jnp.float32),
            pltpu.VMEM((16,), jnp.int32),
            pltpu.SemaphoreType.DMA,
            pltpu.SemaphoreType.DMA,
        ],
    )(functools.partial(_sc_body, vsize))
    out = kern(p2.reshape(-1), rng.reshape(-1))
    return out[:, :RPW].reshape(SC_WORKERS * RPW)


@jax.jit
def kernel(p, rng):
    B, T, V = p.shape
    p2 = p.reshape(B * T, V)
    tc = _tc_sample(p2, rng, V)
    sc = _sc_sample(p2, rng, V)
    return jnp.concatenate([tc, sc]).reshape(B, T)


# speedup vs baseline: 6.3257x; 4.8441x over previous
import functools

import jax
import jax.numpy as jnp
from jax.experimental import pallas as pl
from jax.experimental.pallas import tpu as pltpu

ROWS = 64
CW = 1024


def _scan_lanes(x, width, lanes):
    sh = 1
    while sh < width:
        rolled = pltpu.roll(x, sh, axis=1)
        x = x + jnp.where(lanes >= sh, rolled, 0.0)
        sh *= 2
    return x


def _sampler_body(nchunks, vsize, p_ref, rng_ref, out_ref, sums_ref):
    rngv = rng_ref[0]

    for c in range(nchunks):
        w = min(CW, vsize - c * CW)
        chunk = p_ref[:, c * CW:c * CW + w]
        sums_ref[:, c:c + 1] = jnp.sum(chunk, axis=1, keepdims=True)

    sums = sums_ref[...]
    clanes = jax.lax.broadcasted_iota(jnp.int32, (ROWS, 128), 1)
    incl = _scan_lanes(jnp.where(clanes < nchunks, sums, 0.0), 128, clanes)
    below = jnp.logical_and(incl < rngv, clanes < nchunks)
    widths = jnp.minimum(
        jnp.maximum(vsize - clanes * CW, 0), CW)
    base = jnp.sum(jnp.where(below, widths, 0), axis=1, keepdims=True)
    pstar = jnp.sum(jnp.where(below, sums, 0.0), axis=1, keepdims=True)
    cstar = jnp.sum(jnp.where(below, 1, 0), axis=1, keepdims=True)

    cand = jnp.zeros((ROWS, CW), jnp.float32)
    for c in range(nchunks):
        w = min(CW, vsize - c * CW)
        chunk = p_ref[:, c * CW:c * CW + w]
        if w < CW:
            chunk = jnp.concatenate(
                [chunk, jnp.zeros((ROWS, CW - w), jnp.float32)], axis=1)
        cand = jnp.where(cstar == c, chunk, cand)

    lanes = jax.lax.broadcasted_iota(jnp.int32, (ROWS, CW), 1)
    lc = _scan_lanes(cand, CW, lanes)
    cwidth = jnp.where(cstar >= nchunks, 0,
                       jnp.minimum(vsize - cstar * CW, CW))
    valid = jnp.logical_and(lanes < cwidth, pstar + lc < rngv)
    cnt = jnp.sum(jnp.where(valid, 1, 0), axis=1, keepdims=True)
    out_ref[0] = base + cnt


@jax.jit
def kernel(p, rng):
    B, T, V = p.shape
    R = B * T
    nchunks = -(-V // CW)
    p2 = p.reshape(R, V)
    rng3 = rng.reshape(R // ROWS, ROWS, 1)

    out = pl.pallas_call(
        functools.partial(_sampler_body, nchunks, V),
        grid=(R // ROWS,),
        in_specs=[
            pl.BlockSpec((ROWS, V), lambda i: (i, 0)),
            pl.BlockSpec((1, ROWS, 1), lambda i: (i, 0, 0)),
        ],
        out_specs=pl.BlockSpec((1, ROWS, 1), lambda i: (i, 0, 0)),
        out_shape=jax.ShapeDtypeStruct((R // ROWS, ROWS, 1), jnp.int32),
        scratch_shapes=[pltpu.VMEM((ROWS, 128), jnp.float32)],
        compiler_params=pltpu.CompilerParams(
            dimension_semantics=("arbitrary",)),
    )(p2, rng3)
    return out.reshape(B, T)


# speedup vs baseline: 6.8188x; 1.0780x over previous
import functools

import jax
import jax.numpy as jnp
from jax.experimental import pallas as pl
from jax.experimental.pallas import tpu as pltpu

ROWS = 32
CW = 1024


def _scan_lanes(x, width, lanes):
    sh = 1
    while sh < width:
        rolled = pltpu.roll(x, sh, axis=1)
        x = x + jnp.where(lanes >= sh, rolled, 0.0)
        sh *= 2
    return x


def _sampler_body(nchunks, vsize, p_ref, rng_ref, out_ref, sums_ref):
    rngv = rng_ref[0]

    for c in range(nchunks):
        w = min(CW, vsize - c * CW)
        chunk = p_ref[:, c * CW:c * CW + w]
        sums_ref[:, c:c + 1] = jnp.sum(chunk, axis=1, keepdims=True)

    sums = sums_ref[...]
    clanes = jax.lax.broadcasted_iota(jnp.int32, (ROWS, 128), 1)
    incl = _scan_lanes(jnp.where(clanes < nchunks, sums, 0.0), 128, clanes)
    below = jnp.logical_and(incl < rngv, clanes < nchunks)
    widths = jnp.minimum(
        jnp.maximum(vsize - clanes * CW, 0), CW)
    base = jnp.sum(jnp.where(below, widths, 0), axis=1, keepdims=True)
    pstar = jnp.sum(jnp.where(below, sums, 0.0), axis=1, keepdims=True)
    cstar = jnp.sum(jnp.where(below, 1, 0), axis=1, keepdims=True)

    cand = jnp.zeros((ROWS, CW), jnp.float32)
    for c in range(nchunks):
        w = min(CW, vsize - c * CW)
        chunk = p_ref[:, c * CW:c * CW + w]
        if w < CW:
            chunk = jnp.concatenate(
                [chunk, jnp.zeros((ROWS, CW - w), jnp.float32)], axis=1)
        cand = jnp.where(cstar == c, chunk, cand)

    lanes = jax.lax.broadcasted_iota(jnp.int32, (ROWS, CW), 1)
    lc = _scan_lanes(cand, CW, lanes)
    cwidth = jnp.where(cstar >= nchunks, 0,
                       jnp.minimum(vsize - cstar * CW, CW))
    valid = jnp.logical_and(lanes < cwidth, pstar + lc < rngv)
    cnt = jnp.sum(jnp.where(valid, 1, 0), axis=1, keepdims=True)
    out_ref[0] = base + cnt


@jax.jit
def kernel(p, rng):
    B, T, V = p.shape
    R = B * T
    nchunks = -(-V // CW)
    p2 = p.reshape(R, V)
    rng3 = rng.reshape(R // ROWS, ROWS, 1)

    out = pl.pallas_call(
        functools.partial(_sampler_body, nchunks, V),
        grid=(R // ROWS,),
        in_specs=[
            pl.BlockSpec((ROWS, V), lambda i: (i, 0)),
            pl.BlockSpec((1, ROWS, 1), lambda i: (i, 0, 0)),
        ],
        out_specs=pl.BlockSpec((1, ROWS, 1), lambda i: (i, 0, 0)),
        out_shape=jax.ShapeDtypeStruct((R // ROWS, ROWS, 1), jnp.int32),
        scratch_shapes=[pltpu.VMEM((ROWS, 128), jnp.float32)],
        compiler_params=pltpu.CompilerParams(
            dimension_semantics=("arbitrary",)),
    )(p2, rng3)
    return out.reshape(B, T)
